# baseline (speedup 1.0000x reference)
import functools

import jax
import jax.numpy as jnp
from jax import lax
from jax.experimental import pallas as pl
from jax.experimental.pallas import tpu as pltpu
from jax.experimental.pallas import tpu_sc as plsc

BATCH = 16384
NUM_FIELDS = 26
FIELD_DIM = 100000
OUT_DIM = 16
DENSE_DIM = 13

NUM_CORES = 2
NUM_SUBCORES = 16
NUM_WORKERS = NUM_CORES * NUM_SUBCORES
ROWS_PER_WORKER = BATCH // NUM_WORKERS
BLOCK_ROWS = 128
NUM_BLOCKS = ROWS_PER_WORKER // BLOCK_ROWS
IDX_PER_BLOCK = BLOCK_ROWS * NUM_FIELDS
GATHER_ROWS = 128
NUM_GATHERS = IDX_PER_BLOCK // GATHER_ROWS
NUM_SLICES = IDX_PER_BLOCK // 16


def _dense_body(d_ref, w_ref, b_ref, sb_ref, o_ref):
    o_ref[...] = (
        jnp.dot(d_ref[...], w_ref[...], preferred_element_type=jnp.float32)
        + b_ref[...]
        + sb_ref[...]
    )


def _tree_sum(vals):
    while len(vals) > 1:
        nxt = [vals[i] + vals[i + 1] for i in range(0, len(vals) - 1, 2)]
        if len(vals) % 2:
            nxt.append(vals[-1])
        vals = nxt
    return vals[0]


_SC_MESH = plsc.VectorSubcoreMesh(
    core_axis_name="c", subcore_axis_name="s",
    num_cores=NUM_CORES, num_subcores=NUM_SUBCORES,
)


@functools.partial(
    pl.kernel,
    out_type=jax.ShapeDtypeStruct((BATCH, OUT_DIM), jnp.float32),
    mesh=_SC_MESH,
    scratch_types=[
        pltpu.VMEM((IDX_PER_BLOCK,), jnp.int32),
        pltpu.VMEM((IDX_PER_BLOCK, OUT_DIM), jnp.float32),
        pltpu.VMEM((BLOCK_ROWS, OUT_DIM), jnp.float32),
        pltpu.VMEM((BLOCK_ROWS, OUT_DIM), jnp.float32),
        pltpu.SemaphoreType.DMA,
    ],
    compiler_params=pltpu.CompilerParams(use_tc_tiling_on_sc=False),
)
def _sc_gather_sum(sparse_hbm, dout_hbm, emb_hbm, out_hbm,
                   idx1d, rows_v, dens_v, out_v, sem):
    wid = lax.axis_index("s") * NUM_CORES + lax.axis_index("c")
    base0 = wid * ROWS_PER_WORKER

    def block(b, carry):
        base = base0 + b * BLOCK_ROWS
        pltpu.sync_copy(sparse_hbm.at[pl.ds(base * NUM_FIELDS, IDX_PER_BLOCK)],
                        idx1d)
        pltpu.sync_copy(dout_hbm.at[pl.ds(base, BLOCK_ROWS), :], dens_v)

        def addoff(s, c):
            p0 = s * 16
            f = (lax.iota(jnp.int32, 16) + p0) % NUM_FIELDS
            idx1d[pl.ds(p0, 16)] = idx1d[pl.ds(p0, 16)] + f * FIELD_DIM
            return c

        lax.fori_loop(0, NUM_SLICES, addoff, 0)

        copies = []
        for g in range(NUM_GATHERS):
            copies.append(pltpu.async_copy(
                emb_hbm.at[idx1d.at[pl.ds(g * GATHER_ROWS, GATHER_ROWS)]],
                rows_v.at[pl.ds(g * GATHER_ROWS, GATHER_ROWS), :],
                sem,
            ))
        for cp in copies:
            cp.wait()

        def row(r, c):
            s = r * NUM_FIELDS
            vals = [dens_v[r, :]]
            for k in range(NUM_FIELDS):
                vals.append(rows_v[s + k, :])
            out_v[r, :] = jnp.maximum(_tree_sum(vals), 0.0)
            return c

        lax.fori_loop(0, BLOCK_ROWS, row, 0)
        pltpu.sync_copy(out_v, out_hbm.at[pl.ds(base, BLOCK_ROWS), :])
        return carry

    lax.fori_loop(0, NUM_BLOCKS, block, 0)


def kernel(dense, sparse, W_dense, b_dense, emb_table, sparse_bias):
    dense_out = pl.pallas_call(
        _dense_body,
        out_shape=jax.ShapeDtypeStruct((BATCH, OUT_DIM), jnp.float32),
    )(dense, W_dense.T, b_dense.reshape(1, OUT_DIM), sparse_bias)

    sparse_flat = sparse.astype(jnp.int32).reshape(BATCH * NUM_FIELDS)
    return _sc_gather_sum(sparse_flat, dense_out, emb_table)
